# Initial kernel scaffold; baseline (speedup 1.0000x reference)
import jax
import jax.numpy as jnp
from jax.experimental import pallas as pl


def kernel(x_q, x_t, u_idx, v_idx, v_corres, g_emb, W_enc, b_enc, bilinear_mat, W_dec, b_dec):
    raise NotImplementedError("write your pallas kernel here")


# speedup vs baseline: 7.8793x; 7.8793x over previous
import functools

import jax
import jax.numpy as jnp
from jax import lax
from jax.experimental import pallas as pl
from jax.experimental.pallas import tpu as pltpu
from jax.experimental.pallas import tpu_sc as plsc

G = 4096
N = 65536
V = 131072
F = 64
D_IN = 16
D_OUT = 8

NC = 2
NS = 16
NW = NC * NS
L = 16

_mesh = functools.partial(
    plsc.VectorSubcoreMesh, core_axis_name="c", subcore_axis_name="s"
)


_UG_PER_W = G // NW


def _u_gather_body(xq_hbm, uidx_hbm, out_hbm, idx_v, rows_v, sem):
    wid = lax.axis_index("s") * NC + lax.axis_index("c")
    base = wid * _UG_PER_W
    pltpu.sync_copy(uidx_hbm.at[pl.ds(base, _UG_PER_W)], idx_v)
    pltpu.async_copy(xq_hbm.at[idx_v], rows_v, sem).wait()
    pltpu.sync_copy(rows_v, out_hbm.at[pl.ds(base, _UG_PER_W)])


_u_gather = pl.kernel(
    _u_gather_body,
    out_type=jax.ShapeDtypeStruct((G, F), jnp.float32),
    mesh=_mesh(),
    compiler_params=pltpu.CompilerParams(use_tc_tiling_on_sc=False,
                                        needs_layout_passes=False),
    scratch_types=[
        pltpu.VMEM((_UG_PER_W,), jnp.int32),
        pltpu.VMEM((_UG_PER_W, F), jnp.float32),
        pltpu.SemaphoreType.DMA,
    ],
)


_XT_BLK = 8192


def _dense_body(xt_ref, u_ref, g_ref, wenc_ref, benc_ref, bil_ref, wsel_ref,
                w2_ref, bdec_ref, xtlat_ref, qv_ref, s_ref):
    i = pl.program_id(0)
    xtlat_ref[...] = jax.nn.relu(
        jnp.dot(xt_ref[...], wenc_ref[...], preferred_element_type=jnp.float32)
        + benc_ref[...]
    )

    @pl.when(i == 0)
    def _():
        C = jnp.dot(bil_ref[...], wsel_ref[...],
                    preferred_element_type=jnp.float32)
        lat_u = jax.nn.relu(
            jnp.dot(u_ref[...], wenc_ref[...],
                    preferred_element_type=jnp.float32) + benc_ref[...]
        )
        qv_ref[...] = jnp.dot(lat_u, C, preferred_element_type=jnp.float32)
        s_ref[...] = (
            jnp.dot(g_ref[...], w2_ref[...], preferred_element_type=jnp.float32)
            + bdec_ref[...]
        )


def _dense(x_t, u_rows, g_emb, W_enc, benc2, bil_r, wsel, w2, bdec2):
    nblk = N // _XT_BLK
    return pl.pallas_call(
        _dense_body,
        grid=(nblk,),
        in_specs=[
            pl.BlockSpec((_XT_BLK, F), lambda i: (i, 0)),
            pl.BlockSpec((G, F), lambda i: (0, 0)),
            pl.BlockSpec((G, D_IN), lambda i: (0, 0)),
            pl.BlockSpec((F, D_IN), lambda i: (0, 0)),
            pl.BlockSpec((1, D_IN), lambda i: (0, 0)),
            pl.BlockSpec((D_IN, D_IN * D_OUT), lambda i: (0, 0)),
            pl.BlockSpec((D_IN * D_OUT, D_IN), lambda i: (0, 0)),
            pl.BlockSpec((D_IN, 1), lambda i: (0, 0)),
            pl.BlockSpec((1, 1), lambda i: (0, 0)),
        ],
        out_specs=[
            pl.BlockSpec((_XT_BLK, D_IN), lambda i: (i, 0)),
            pl.BlockSpec((G, D_IN), lambda i: (0, 0)),
            pl.BlockSpec((G, 1), lambda i: (0, 0)),
        ],
        out_shape=[
            jax.ShapeDtypeStruct((N, D_IN), jnp.float32),
            jax.ShapeDtypeStruct((G, D_IN), jnp.float32),
            jax.ShapeDtypeStruct((G, 1), jnp.float32),
        ],
    )(x_t, u_rows, g_emb, W_enc, benc2, bil_r, wsel, w2, bdec2)


_R_PER_W = V // NW
_SUB = 128
_NSUB = _R_PER_W // _SUB


def _main_body(xt_hbm, qv_hbm, s_hbm, vidx_hbm, vcor_hbm, out_hbm,
               qv_v, s_v, idx_v, cor_v, rows_v, out_v, sem):
    wid = lax.axis_index("s") * NC + lax.axis_index("c")
    base = wid * _R_PER_W
    pltpu.sync_copy(qv_hbm, qv_v)
    pltpu.sync_copy(s_hbm, s_v)
    pltpu.sync_copy(vidx_hbm.at[pl.ds(base, _R_PER_W)], idx_v)
    pltpu.sync_copy(vcor_hbm.at[pl.ds(base, _R_PER_W)], cor_v)

    lane = jnp.arange(L, dtype=jnp.int32)

    def sub(j, carry):
        off = j * _SUB
        pltpu.async_copy(xt_hbm.at[idx_v.at[pl.ds(off, _SUB)]], rows_v,
                         sem).wait()
        for t in range(_SUB // L):
            r0 = t * L
            g16 = cor_v[pl.ds(off + r0, L)]
            acc = plsc.load_gather(s_v, [g16])
            rows16 = lane + r0
            for d in range(D_IN):
                d16 = jnp.full((L,), d, jnp.int32)
                xt_d = plsc.load_gather(rows_v, [rows16, d16])
                q_d = plsc.load_gather(qv_v, [g16, d16])
                acc = acc + xt_d * q_d
            out_v[pl.ds(off + r0, L)] = acc
        return carry

    lax.fori_loop(0, _NSUB, sub, 0)
    pltpu.sync_copy(out_v, out_hbm.at[pl.ds(base, _R_PER_W)])


_main = pl.kernel(
    _main_body,
    out_type=jax.ShapeDtypeStruct((V,), jnp.float32),
    mesh=_mesh(),
    compiler_params=pltpu.CompilerParams(use_tc_tiling_on_sc=False,
                                        needs_layout_passes=False),
    scratch_types=[
        pltpu.VMEM((G, D_IN), jnp.float32),
        pltpu.VMEM((G,), jnp.float32),
        pltpu.VMEM((_R_PER_W,), jnp.int32),
        pltpu.VMEM((_R_PER_W,), jnp.int32),
        pltpu.VMEM((_SUB, D_IN), jnp.float32),
        pltpu.VMEM((_R_PER_W,), jnp.float32),
        pltpu.SemaphoreType.DMA,
    ],
)


def kernel(x_q, x_t, u_idx, v_idx, v_corres, g_emb, W_enc, b_enc,
           bilinear_mat, W_dec, b_dec):
    bil_r = bilinear_mat.reshape(D_IN, D_IN * D_OUT)
    wsel = jnp.kron(jnp.eye(D_IN, dtype=jnp.float32), W_dec[:D_OUT])
    w2 = W_dec[D_OUT:]
    benc2 = b_enc.reshape(1, D_IN)
    bdec2 = b_dec.reshape(1, 1)

    u_rows = _u_gather(x_q, u_idx)
    xt_lat, qv, s = _dense(x_t, u_rows, g_emb, W_enc, benc2, bil_r, wsel, w2,
                           bdec2)
    return _main(xt_lat, qv, s.reshape(G), v_idx, v_corres)
